# Initial kernel scaffold; baseline (speedup 1.0000x reference)
import jax
import jax.numpy as jnp
from jax.experimental import pallas as pl


def kernel(node_scalar, batch, W1, b1, W2, b2):
    raise NotImplementedError("write your pallas kernel here")


# speedup vs baseline: 1.9815x; 1.9815x over previous
import functools

import jax
import jax.numpy as jnp
from jax import lax
from jax.experimental import pallas as pl

G = 1024
G_HI = 8
G_LO = 128
BN = 2000


def _body(x_ref, idx_ref, w1_ref, b1_ref, w2_ref, b2_ref, out_ref):
    i = pl.program_id(0)

    x = x_ref[...]
    h = jnp.dot(x, w1_ref[...], preferred_element_type=jnp.float32)
    h = h + b1_ref[...]
    h = h * jax.nn.sigmoid(h)
    e = jnp.dot(h, w2_ref[...], preferred_element_type=jnp.float32)
    e = e + b2_ref[...]

    g = idx_ref[0, 0, :].reshape(BN, 1)
    hi = g // G_LO
    lo = g - hi * G_LO
    iota_hi = lax.broadcasted_iota(jnp.int32, (BN, G_HI), 1)
    iota_lo = lax.broadcasted_iota(jnp.int32, (BN, G_LO), 1)
    a = jnp.where(hi == iota_hi, e, 0.0)
    b = (lo == iota_lo).astype(jnp.float32)
    contrib = lax.dot_general(a, b, (((0,), (0,)), ((), ())),
                              preferred_element_type=jnp.float32)

    @pl.when(i == 0)
    def _():
        out_ref[...] = jnp.zeros_like(out_ref)

    out_ref[...] += contrib


def kernel(node_scalar, batch, W1, b1, W2, b2):
    n, d = node_scalar.shape
    hdim = W1.shape[1]
    num_blocks = n // BN
    assert num_blocks * BN == n

    idx3 = batch.astype(jnp.int32).reshape(num_blocks, 1, BN)

    out = pl.pallas_call(
        _body,
        grid=(num_blocks,),
        in_specs=[
            pl.BlockSpec((BN, d), lambda i: (i, 0)),
            pl.BlockSpec((1, 1, BN), lambda i: (i, 0, 0)),
            pl.BlockSpec((d, hdim), lambda i: (0, 0)),
            pl.BlockSpec((1, hdim), lambda i: (0, 0)),
            pl.BlockSpec((hdim, 1), lambda i: (0, 0)),
            pl.BlockSpec((1, 1), lambda i: (0, 0)),
        ],
        out_specs=pl.BlockSpec((G_HI, G_LO), lambda i: (0, 0)),
        out_shape=jax.ShapeDtypeStruct((G_HI, G_LO), jnp.float32),
    )(node_scalar, idx3, W1, b1.reshape(1, hdim), W2, b2.reshape(1, 1))

    return out.reshape(G)


# speedup vs baseline: 1.9966x; 1.0076x over previous
import functools

import jax
import jax.numpy as jnp
from jax import lax
from jax.experimental import pallas as pl
from jax.experimental.pallas import tpu as pltpu
from jax.experimental.pallas import tpu_sc as plsc

G = 1024
N_NODES = 100000
BN = 2000
NUM_BLOCKS = N_NODES // BN

NC = 2
NS = 16
NW = NC * NS
CHUNK = 128
NCHUNK = 25
BW = CHUNK * NCHUNK
NPAD = BW * NW


def _mlp_body(x_ref, w1_ref, b1_ref, w2_ref, b2_ref, out_ref):
    x = x_ref[...]
    h = jnp.dot(x, w1_ref[...], preferred_element_type=jnp.float32)
    h = h + b1_ref[...]
    h = 0.5 * h * (1.0 + lax.tanh(0.5 * h))
    e = lax.dot_general(w2_ref[...], h, (((0,), (1,)), ((), ())),
                        preferred_element_type=jnp.float32)
    out_ref[0] = e + b2_ref[...]


_mesh = plsc.VectorSubcoreMesh(core_axis_name="c", subcore_axis_name="s")


@functools.partial(
    pl.kernel,
    mesh=_mesh,
    out_type=jax.ShapeDtypeStruct((NC, G), jnp.float32),
    scratch_types=[
        pltpu.VMEM((NCHUNK, CHUNK), jnp.float32),
        pltpu.VMEM((NCHUNK, CHUNK), jnp.int32),
        pltpu.VMEM_SHARED((G,), jnp.float32),
    ],
)
def _segsum(e_hbm, idx_hbm, zeros_hbm, out_hbm, e_v, idx_v, acc_sh):
    c = lax.axis_index("c")
    s = lax.axis_index("s")
    wid = s * NC + c

    pltpu.sync_copy(e_hbm.at[wid], e_v)
    pltpu.sync_copy(idx_hbm.at[wid], idx_v)

    @pl.when(s == 0)
    def _():
        pltpu.sync_copy(zeros_hbm, acc_sh)

    plsc.subcore_barrier()

    def body(j, carry):
        pltpu.sync_copy(e_v.at[j], acc_sh.at[idx_v.at[j]], add=True)
        return carry

    lax.fori_loop(0, NCHUNK, body, 0)

    plsc.subcore_barrier()

    @pl.when(s == 0)
    def _():
        pltpu.sync_copy(acc_sh, out_hbm.at[c])


def kernel(node_scalar, batch, W1, b1, W2, b2):
    n, d = node_scalar.shape
    hdim = W1.shape[1]
    assert n == N_NODES

    e = pl.pallas_call(
        _mlp_body,
        grid=(NUM_BLOCKS,),
        in_specs=[
            pl.BlockSpec((BN, d), lambda i: (i, 0)),
            pl.BlockSpec((d, hdim), lambda i: (0, 0)),
            pl.BlockSpec((1, hdim), lambda i: (0, 0)),
            pl.BlockSpec((hdim, 1), lambda i: (0, 0)),
            pl.BlockSpec((1, 1), lambda i: (0, 0)),
        ],
        out_specs=pl.BlockSpec((1, 1, BN), lambda i: (i, 0, 0)),
        out_shape=jax.ShapeDtypeStruct((NUM_BLOCKS, 1, BN), jnp.float32),
    )(node_scalar, W1, b1.reshape(1, hdim), W2, b2.reshape(1, 1))

    e_pad = jnp.concatenate(
        [e.reshape(n), jnp.zeros((NPAD - n,), jnp.float32)])
    ids_pad = jnp.concatenate(
        [batch.astype(jnp.int32), jnp.zeros((NPAD - n,), jnp.int32)])
    e3 = e_pad.reshape(NW, NCHUNK, CHUNK)
    idx3 = ids_pad.reshape(NW, NCHUNK, CHUNK)

    partials = _segsum(e3, idx3, jnp.zeros((G,), jnp.float32))
    return partials[0] + partials[1]


# speedup vs baseline: 2.3628x; 1.1834x over previous
import functools

import jax
import jax.numpy as jnp
from jax import lax
from jax.experimental import pallas as pl
from jax.experimental.pallas import tpu as pltpu
from jax.experimental.pallas import tpu_sc as plsc

G = 1024
N_NODES = 100000
BN = 5000
NUM_BLOCKS = N_NODES // BN

NC = 2
NS = 16
NW = NC * NS
CHUNK = 128
NCHUNK = 25
BW = CHUNK * NCHUNK
NPAD = BW * NW


def _mlp_body(x_ref, w1_ref, b1_ref, w2_ref, b2_ref, out_ref):
    x = x_ref[...]
    h = jnp.dot(x, w1_ref[...], preferred_element_type=jnp.float32)
    h = h + b1_ref[...]
    h = 0.5 * h * (1.0 + lax.tanh(0.5 * h))
    e = lax.dot_general(w2_ref[...], h, (((0,), (1,)), ((), ())),
                        preferred_element_type=jnp.float32)
    out_ref[0] = e + b2_ref[...]


_mesh = plsc.VectorSubcoreMesh(core_axis_name="c", subcore_axis_name="s")


@functools.partial(
    pl.kernel,
    mesh=_mesh,
    out_type=jax.ShapeDtypeStruct((NC, G), jnp.float32),
    scratch_types=[
        pltpu.VMEM((NCHUNK, CHUNK), jnp.float32),
        pltpu.VMEM((NCHUNK, CHUNK), jnp.int32),
        pltpu.VMEM_SHARED((G,), jnp.float32),
    ],
)
def _segsum(e_hbm, idx_hbm, zeros_hbm, out_hbm, e_v, idx_v, acc_sh):
    c = lax.axis_index("c")
    s = lax.axis_index("s")
    wid = s * NC + c

    pltpu.sync_copy(e_hbm.at[wid], e_v)
    pltpu.sync_copy(idx_hbm.at[wid], idx_v)

    @pl.when(s == 0)
    def _():
        pltpu.sync_copy(zeros_hbm, acc_sh)

    plsc.subcore_barrier()

    def body(j, carry):
        pltpu.sync_copy(e_v.at[j], acc_sh.at[idx_v.at[j]], add=True)
        return carry

    lax.fori_loop(0, NCHUNK, body, 0)

    plsc.subcore_barrier()

    @pl.when(s == 0)
    def _():
        pltpu.sync_copy(acc_sh, out_hbm.at[c])


def kernel(node_scalar, batch, W1, b1, W2, b2):
    n, d = node_scalar.shape
    hdim = W1.shape[1]
    assert n == N_NODES

    e = pl.pallas_call(
        _mlp_body,
        grid=(NUM_BLOCKS,),
        in_specs=[
            pl.BlockSpec((BN, d), lambda i: (i, 0)),
            pl.BlockSpec((d, hdim), lambda i: (0, 0)),
            pl.BlockSpec((1, hdim), lambda i: (0, 0)),
            pl.BlockSpec((hdim, 1), lambda i: (0, 0)),
            pl.BlockSpec((1, 1), lambda i: (0, 0)),
        ],
        out_specs=pl.BlockSpec((1, 1, BN), lambda i: (i, 0, 0)),
        out_shape=jax.ShapeDtypeStruct((NUM_BLOCKS, 1, BN), jnp.float32),
    )(node_scalar, W1, b1.reshape(1, hdim), W2, b2.reshape(1, 1))

    e_pad = jnp.concatenate(
        [e.reshape(n), jnp.zeros((NPAD - n,), jnp.float32)])
    ids_pad = jnp.concatenate(
        [batch.astype(jnp.int32), jnp.zeros((NPAD - n,), jnp.int32)])
    e3 = e_pad.reshape(NW, NCHUNK, CHUNK)
    idx3 = ids_pad.reshape(NW, NCHUNK, CHUNK)

    partials = _segsum(e3, idx3, jnp.zeros((G,), jnp.float32))
    return partials[0] + partials[1]


# speedup vs baseline: 2.8585x; 1.2098x over previous
import functools

import jax
import jax.numpy as jnp
from jax import lax
from jax.experimental import pallas as pl
from jax.experimental.pallas import tpu as pltpu
from jax.experimental.pallas import tpu_sc as plsc

G = 1024
N_NODES = 100000
BN = 10000
NUM_BLOCKS = N_NODES // BN

NC = 2
NS = 16
NW = NC * NS
CHUNK = 128
NCHUNK = 25
BW = CHUNK * NCHUNK
NPAD = BW * NW


def _mlp_body(x_ref, w1_ref, b1_ref, w2_ref, b2_ref, out_ref):
    x = x_ref[...]
    h = jnp.dot(x, w1_ref[...], preferred_element_type=jnp.float32)
    h = h + b1_ref[...]
    h = 0.5 * h * (1.0 + lax.tanh(0.5 * h))
    e = lax.dot_general(w2_ref[...], h, (((0,), (1,)), ((), ())),
                        preferred_element_type=jnp.float32)
    out_ref[0] = e + b2_ref[...]


_mesh = plsc.VectorSubcoreMesh(core_axis_name="c", subcore_axis_name="s")


@functools.partial(
    pl.kernel,
    mesh=_mesh,
    out_type=jax.ShapeDtypeStruct((NC, G), jnp.float32),
    scratch_types=[
        pltpu.VMEM((NCHUNK, CHUNK), jnp.float32),
        pltpu.VMEM((NCHUNK, CHUNK), jnp.int32),
        pltpu.VMEM_SHARED((G,), jnp.float32),
    ],
)
def _segsum(e_hbm, idx_hbm, zeros_hbm, out_hbm, e_v, idx_v, acc_sh):
    c = lax.axis_index("c")
    s = lax.axis_index("s")
    wid = s * NC + c

    pltpu.sync_copy(e_hbm.at[wid], e_v)
    pltpu.sync_copy(idx_hbm.at[wid], idx_v)

    @pl.when(s == 0)
    def _():
        pltpu.sync_copy(zeros_hbm, acc_sh)

    plsc.subcore_barrier()

    def body(j, carry):
        pltpu.sync_copy(e_v.at[j], acc_sh.at[idx_v.at[j]], add=True)
        return carry

    lax.fori_loop(0, NCHUNK, body, 0)

    plsc.subcore_barrier()

    @pl.when(s == 0)
    def _():
        pltpu.sync_copy(acc_sh, out_hbm.at[c])


def kernel(node_scalar, batch, W1, b1, W2, b2):
    n, d = node_scalar.shape
    hdim = W1.shape[1]
    assert n == N_NODES

    e = pl.pallas_call(
        _mlp_body,
        grid=(NUM_BLOCKS,),
        in_specs=[
            pl.BlockSpec((BN, d), lambda i: (i, 0)),
            pl.BlockSpec((d, hdim), lambda i: (0, 0)),
            pl.BlockSpec((1, hdim), lambda i: (0, 0)),
            pl.BlockSpec((hdim, 1), lambda i: (0, 0)),
            pl.BlockSpec((1, 1), lambda i: (0, 0)),
        ],
        out_specs=pl.BlockSpec((1, 1, BN), lambda i: (i, 0, 0)),
        out_shape=jax.ShapeDtypeStruct((NUM_BLOCKS, 1, BN), jnp.float32),
    )(node_scalar, W1, b1.reshape(1, hdim), W2, b2.reshape(1, 1))

    e_pad = jnp.concatenate(
        [e.reshape(n), jnp.zeros((NPAD - n,), jnp.float32)])
    ids_pad = jnp.concatenate(
        [batch.astype(jnp.int32), jnp.zeros((NPAD - n,), jnp.int32)])
    e3 = e_pad.reshape(NW, NCHUNK, CHUNK)
    idx3 = ids_pad.reshape(NW, NCHUNK, CHUNK)

    partials = _segsum(e3, idx3, jnp.zeros((G,), jnp.float32))
    return partials[0] + partials[1]


# speedup vs baseline: 2.9438x; 1.0298x over previous
import functools

import jax
import jax.numpy as jnp
from jax import lax
from jax.experimental import pallas as pl
from jax.experimental.pallas import tpu as pltpu
from jax.experimental.pallas import tpu_sc as plsc

G = 1024
N_NODES = 100000
BN = 20000
NUM_BLOCKS = N_NODES // BN

NC = 2
NS = 16
NW = NC * NS
CHUNK = 128
NCHUNK = 25
BW = CHUNK * NCHUNK
NPAD = BW * NW


def _mlp_body(x_ref, w1_ref, b1_ref, w2_ref, b2_ref, out_ref):
    x = x_ref[...]
    h = jnp.dot(x, w1_ref[...], preferred_element_type=jnp.float32)
    h = h + b1_ref[...]
    h = 0.5 * h * (1.0 + lax.tanh(0.5 * h))
    e = lax.dot_general(w2_ref[...], h, (((0,), (1,)), ((), ())),
                        preferred_element_type=jnp.float32)
    out_ref[0] = e + b2_ref[...]


_mesh = plsc.VectorSubcoreMesh(core_axis_name="c", subcore_axis_name="s")


@functools.partial(
    pl.kernel,
    mesh=_mesh,
    out_type=jax.ShapeDtypeStruct((NC, G), jnp.float32),
    scratch_types=[
        pltpu.VMEM((NCHUNK, CHUNK), jnp.float32),
        pltpu.VMEM((NCHUNK, CHUNK), jnp.int32),
        pltpu.VMEM_SHARED((G,), jnp.float32),
    ],
)
def _segsum(e_hbm, idx_hbm, zeros_hbm, out_hbm, e_v, idx_v, acc_sh):
    c = lax.axis_index("c")
    s = lax.axis_index("s")
    wid = s * NC + c

    pltpu.sync_copy(e_hbm.at[wid], e_v)
    pltpu.sync_copy(idx_hbm.at[wid], idx_v)

    @pl.when(s == 0)
    def _():
        pltpu.sync_copy(zeros_hbm, acc_sh)

    plsc.subcore_barrier()

    def body(j, carry):
        pltpu.sync_copy(e_v.at[j], acc_sh.at[idx_v.at[j]], add=True)
        return carry

    lax.fori_loop(0, NCHUNK, body, 0)

    plsc.subcore_barrier()

    @pl.when(s == 0)
    def _():
        pltpu.sync_copy(acc_sh, out_hbm.at[c])


def kernel(node_scalar, batch, W1, b1, W2, b2):
    n, d = node_scalar.shape
    hdim = W1.shape[1]
    assert n == N_NODES

    e = pl.pallas_call(
        _mlp_body,
        grid=(NUM_BLOCKS,),
        in_specs=[
            pl.BlockSpec((BN, d), lambda i: (i, 0)),
            pl.BlockSpec((d, hdim), lambda i: (0, 0)),
            pl.BlockSpec((1, hdim), lambda i: (0, 0)),
            pl.BlockSpec((hdim, 1), lambda i: (0, 0)),
            pl.BlockSpec((1, 1), lambda i: (0, 0)),
        ],
        out_specs=pl.BlockSpec((1, 1, BN), lambda i: (i, 0, 0)),
        out_shape=jax.ShapeDtypeStruct((NUM_BLOCKS, 1, BN), jnp.float32),
    )(node_scalar, W1, b1.reshape(1, hdim), W2, b2.reshape(1, 1))

    e_pad = jnp.concatenate(
        [e.reshape(n), jnp.zeros((NPAD - n,), jnp.float32)])
    ids_pad = jnp.concatenate(
        [batch.astype(jnp.int32), jnp.zeros((NPAD - n,), jnp.int32)])
    e3 = e_pad.reshape(NW, NCHUNK, CHUNK)
    idx3 = ids_pad.reshape(NW, NCHUNK, CHUNK)

    partials = _segsum(e3, idx3, jnp.zeros((G,), jnp.float32))
    return partials[0] + partials[1]
